# baseline (speedup 1.0000x reference)
import jax
import jax.numpy as jnp
from jax.experimental import pallas as pl
from jax.experimental.pallas import tpu as pltpu

_BANK = 100000
_DIM = 128
_BATCH = 16384
_TAIL = _BANK - _BATCH


def _body(feat_ref, bank_ref, lab_ref, lbank_ref, out_fb_ref, out_lb_ref,
          sem_fb_head, sem_fb_tail, sem_lb_head, sem_lb_tail):
    c0 = pltpu.make_async_copy(
        feat_ref, out_fb_ref.at[pl.ds(0, _BATCH), :], sem_fb_head)
    c1 = pltpu.make_async_copy(
        bank_ref.at[pl.ds(_BATCH, _TAIL), :],
        out_fb_ref.at[pl.ds(_BATCH, _TAIL), :], sem_fb_tail)
    c2 = pltpu.make_async_copy(
        lab_ref, out_lb_ref.at[pl.ds(0, _BATCH)], sem_lb_head)
    c3 = pltpu.make_async_copy(
        lbank_ref.at[pl.ds(_BATCH, _TAIL)],
        out_lb_ref.at[pl.ds(_BATCH, _TAIL)], sem_lb_tail)
    c0.start()
    c1.start()
    c2.start()
    c3.start()
    c0.wait()
    c1.wait()
    c2.wait()
    c3.wait()


def kernel(features, labels, feature_bank, label_bank):
    any_spec = pl.BlockSpec(memory_space=pl.ANY)
    out_fb, out_lb = pl.pallas_call(
        _body,
        in_specs=[any_spec, any_spec, any_spec, any_spec],
        out_specs=[any_spec, any_spec],
        out_shape=[
            jax.ShapeDtypeStruct((_BANK, _DIM), feature_bank.dtype),
            jax.ShapeDtypeStruct((_BANK,), label_bank.dtype),
        ],
        scratch_shapes=[pltpu.SemaphoreType.DMA] * 4,
    )(features, feature_bank, labels, label_bank)
    return out_fb, out_lb


# speedup vs baseline: 30.1997x; 30.1997x over previous
import jax
import jax.numpy as jnp
from jax.experimental import pallas as pl

_BANK = 100000
_DIM = 128
_BATCH = 16384
_BLK = 2048
_NB = (_BANK + _BLK - 1) // _BLK
_SPLIT = _BATCH // _BLK


def _body(feat_ref, bank_ref, lab_ref, lbank_ref, out_fb_ref, out_lb_ref):
    i = pl.program_id(0)

    @pl.when(i < _SPLIT)
    def _():
        out_fb_ref[...] = feat_ref[...]

    @pl.when(i >= _SPLIT)
    def _():
        out_fb_ref[...] = bank_ref[...]

    @pl.when(i == 0)
    def _():
        out_lb_ref[0:_BATCH] = lab_ref[...]
        out_lb_ref[_BATCH:_BANK] = lbank_ref[_BATCH:_BANK]


def kernel(features, labels, feature_bank, label_bank):
    out_fb, out_lb = pl.pallas_call(
        _body,
        grid=(_NB,),
        in_specs=[
            pl.BlockSpec((_BLK, _DIM), lambda i: (jnp.minimum(i, _SPLIT - 1), 0)),
            pl.BlockSpec((_BLK, _DIM), lambda i: (jnp.maximum(i, _SPLIT), 0)),
            pl.BlockSpec((_BATCH,), lambda i: (0,)),
            pl.BlockSpec((_BANK,), lambda i: (0,)),
        ],
        out_specs=[
            pl.BlockSpec((_BLK, _DIM), lambda i: (i, 0)),
            pl.BlockSpec((_BANK,), lambda i: (0,)),
        ],
        out_shape=[
            jax.ShapeDtypeStruct((_BANK, _DIM), feature_bank.dtype),
            jax.ShapeDtypeStruct((_BANK,), label_bank.dtype),
        ],
    )(features, feature_bank, labels, label_bank)
    return out_fb, out_lb


# speedup vs baseline: 41.5899x; 1.3772x over previous
import jax
import jax.numpy as jnp
from jax.experimental import pallas as pl

_BANK = 100000
_DIM = 128
_BATCH = 16384
_BLK = 4096
_NB = (_BANK + _BLK - 1) // _BLK
_SPLIT = _BATCH // _BLK


def _body(feat_ref, bank_ref, lab_ref, lbank_ref, out_fb_ref, out_lb_ref):
    i = pl.program_id(0)

    @pl.when(i < _SPLIT)
    def _():
        out_fb_ref[...] = feat_ref[...]

    @pl.when(i >= _SPLIT)
    def _():
        out_fb_ref[...] = bank_ref[...]

    @pl.when(i == 0)
    def _():
        out_lb_ref[0:_BATCH] = lab_ref[...]
        out_lb_ref[_BATCH:_BANK] = lbank_ref[_BATCH:_BANK]


def kernel(features, labels, feature_bank, label_bank):
    out_fb, out_lb = pl.pallas_call(
        _body,
        grid=(_NB,),
        in_specs=[
            pl.BlockSpec((_BLK, _DIM), lambda i: (jnp.minimum(i, _SPLIT - 1), 0)),
            pl.BlockSpec((_BLK, _DIM), lambda i: (jnp.maximum(i, _SPLIT), 0)),
            pl.BlockSpec((_BATCH,), lambda i: (0,)),
            pl.BlockSpec((_BANK,), lambda i: (0,)),
        ],
        out_specs=[
            pl.BlockSpec((_BLK, _DIM), lambda i: (i, 0)),
            pl.BlockSpec((_BANK,), lambda i: (0,)),
        ],
        out_shape=[
            jax.ShapeDtypeStruct((_BANK, _DIM), feature_bank.dtype),
            jax.ShapeDtypeStruct((_BANK,), label_bank.dtype),
        ],
    )(features, feature_bank, labels, label_bank)
    return out_fb, out_lb


# speedup vs baseline: 46.7062x; 1.1230x over previous
import jax
import jax.numpy as jnp
from jax.experimental import pallas as pl

_BANK = 100000
_DIM = 128
_BATCH = 16384
_BLK = 8192
_NB = (_BANK + _BLK - 1) // _BLK
_SPLIT = _BATCH // _BLK


def _body(feat_ref, bank_ref, lab_ref, lbank_ref, out_fb_ref, out_lb_ref):
    i = pl.program_id(0)

    @pl.when(i < _SPLIT)
    def _():
        out_fb_ref[...] = feat_ref[...]

    @pl.when(i >= _SPLIT)
    def _():
        out_fb_ref[...] = bank_ref[...]

    @pl.when(i == 0)
    def _():
        out_lb_ref[0:_BATCH] = lab_ref[...]
        out_lb_ref[_BATCH:_BANK] = lbank_ref[_BATCH:_BANK]


def kernel(features, labels, feature_bank, label_bank):
    out_fb, out_lb = pl.pallas_call(
        _body,
        grid=(_NB,),
        in_specs=[
            pl.BlockSpec((_BLK, _DIM), lambda i: (jnp.minimum(i, _SPLIT - 1), 0)),
            pl.BlockSpec((_BLK, _DIM), lambda i: (jnp.maximum(i, _SPLIT), 0)),
            pl.BlockSpec((_BATCH,), lambda i: (0,)),
            pl.BlockSpec((_BANK,), lambda i: (0,)),
        ],
        out_specs=[
            pl.BlockSpec((_BLK, _DIM), lambda i: (i, 0)),
            pl.BlockSpec((_BANK,), lambda i: (0,)),
        ],
        out_shape=[
            jax.ShapeDtypeStruct((_BANK, _DIM), feature_bank.dtype),
            jax.ShapeDtypeStruct((_BANK,), label_bank.dtype),
        ],
    )(features, feature_bank, labels, label_bank)
    return out_fb, out_lb
